# Initial kernel scaffold; baseline (speedup 1.0000x reference)
import jax
import jax.numpy as jnp
from jax.experimental import pallas as pl


def kernel(hn, he, edge_index, norm, W1a, b1a, W1b, b1b, W2a, b2a, W2b, b2b):
    raise NotImplementedError("write your pallas kernel here")


# speedup vs baseline: 2.1382x; 2.1382x over previous
import functools

import jax
import jax.numpy as jnp
from jax import lax
from jax.experimental import pallas as pl
from jax.experimental.pallas import tpu as pltpu
from jax.experimental.pallas import tpu_sc as plsc

_BF = jnp.bfloat16
_F32 = jnp.float32

_GW = 128
_SW = 128


def _sc_gather(table, idx2):
    num_idx = idx2.shape[1]
    d = table.shape[1]

    @functools.partial(
        pl.kernel,
        out_type=jax.ShapeDtypeStruct((num_idx, d), table.dtype),
        mesh=plsc.VectorSubcoreMesh(core_axis_name="c", subcore_axis_name="s"),
    )
    def gat(tbl_hbm, i_hbm, o_hbm):
        def body(i_vmem, o_vmem):
            pltpu.sync_copy(tbl_hbm.at[i_vmem.at[0]], o_vmem)

        pltpu.emit_pipeline(
            body,
            grid=(num_idx // _GW,),
            in_specs=[pl.BlockSpec((1, _GW), lambda i: (0, i))],
            out_specs=[pl.BlockSpec((_GW, d), lambda i: (i, 0))],
            core_axis_name=("c", "s"),
            dimension_semantics=(pltpu.PARALLEL,),
        )(i_hbm, o_hbm)

    return gat(table, idx2)


def _sc_scatter_add(weighted, dst2, out_ref):
    e_total, d = weighted.shape
    nchunks = e_total // _SW

    @functools.partial(
        pl.kernel,
        out_type=(),
        mesh=plsc.VectorSubcoreMesh(core_axis_name="c", subcore_axis_name="s"),
        scratch_types=[
            pltpu.VMEM((1, _SW), jnp.int32),
            pltpu.VMEM((_SW, d), _F32),
        ],
    )
    def scat(w_hbm, dst_hbm, o_hbm, idxraw, rowbuf):
        c = lax.axis_index("c")
        s = lax.axis_index("s")
        w = s * 2 + c

        @pl.loop(0, (nchunks + 31) // 32)
        def _(k):
            chunk = k * 32 + w

            @pl.when(chunk < nchunks)
            def _():
                e0 = chunk * _SW
                pltpu.sync_copy(dst_hbm.at[:, pl.ds(e0, _SW)], idxraw)
                pltpu.sync_copy(w_hbm.at[pl.ds(e0, _SW)], rowbuf)
                pltpu.sync_copy(rowbuf, o_hbm.at[idxraw.at[0]], add=True)

    scat(weighted, dst2, out_ref)


def _edge_body(he_ref, gs_ref, gd_ref, norm_ref, we_ref, ws_ref, wd_ref,
               b1a_ref, w1b_ref, b1b_ref, out_he_ref, out_w_ref):
    he = he_ref[...]
    x = jnp.dot(gs_ref[0].astype(_BF), ws_ref[...], preferred_element_type=_F32)
    x += jnp.dot(gd_ref[0].astype(_BF), wd_ref[...], preferred_element_type=_F32)
    x += jnp.dot(he.astype(_BF), we_ref[...], preferred_element_type=_F32)
    x += b1a_ref[...]
    t = jnp.tanh(x)
    h = jnp.dot(t.astype(_BF), w1b_ref[...], preferred_element_type=_F32)
    he_new = he + (h + b1b_ref[...])
    out_he_ref[...] = he_new
    out_w_ref[...] = he_new * norm_ref[...]


def _tc_edge(he, g3, norm2, we, ws, wd, b1a2, w1b, b1b2):
    e_total, d = he.shape
    be = 640
    row = pl.BlockSpec((be, d), lambda i: (i, 0))
    wspec = pl.BlockSpec((d, d), lambda i: (0, 0))
    bspec = pl.BlockSpec((1, d), lambda i: (0, 0))
    return pl.pallas_call(
        _edge_body,
        grid=(e_total // be,),
        in_specs=[
            row,
            pl.BlockSpec((1, be, d), lambda i: (0, i, 0)),
            pl.BlockSpec((1, be, d), lambda i: (1, i, 0)),
            pl.BlockSpec((be, 1), lambda i: (i, 0)),
            wspec, wspec, wspec, bspec, wspec, bspec,
        ],
        out_specs=[row, row],
        out_shape=[jax.ShapeDtypeStruct((e_total, d), _F32)] * 2,
    )(he, g3, g3, norm2, we, ws, wd, b1a2, w1b, b1b2)


def _node_body(hn_ref, nf_ref, wh_ref, wf_ref, b2a_ref, w2b_ref, b2b_ref,
               out_ref):
    hn = hn_ref[...]
    x = jnp.dot(hn.astype(_BF), wh_ref[...], preferred_element_type=_F32)
    x += jnp.dot(nf_ref[...].astype(_BF), wf_ref[...], preferred_element_type=_F32)
    x += b2a_ref[...]
    t = jnp.tanh(x)
    h = jnp.dot(t.astype(_BF), w2b_ref[...], preferred_element_type=_F32)
    out_ref[...] = hn + (h + b2b_ref[...])


def _tc_node(hn, node_ftr, wh, wf, b2a2, w2b, b2b2):
    n, d = hn.shape
    bn = 1000
    row = pl.BlockSpec((bn, d), lambda i: (i, 0))
    wspec = pl.BlockSpec((d, d), lambda i: (0, 0))
    bspec = pl.BlockSpec((1, d), lambda i: (0, 0))
    return pl.pallas_call(
        _node_body,
        grid=(n // bn,),
        in_specs=[row, row, wspec, wspec, bspec, wspec, bspec],
        out_specs=row,
        out_shape=jax.ShapeDtypeStruct((n, d), _F32),
    )(hn, node_ftr, wh, wf, b2a2, w2b, b2b2)


def kernel(hn, he, edge_index, norm, W1a, b1a, W1b, b1b, W2a, b2a, W2b, b2b):
    n, d = hn.shape
    e_total = he.shape[0]
    src = edge_index[0].astype(jnp.int32)
    dst = edge_index[1].astype(jnp.int32)

    idx2 = jnp.concatenate([src, dst]).reshape(1, 2 * e_total)
    gathered = _sc_gather(hn, idx2)
    g3 = gathered.reshape(2, e_total, d)

    we = W1a[:d].astype(_BF)
    ws = W1a[d:2 * d].astype(_BF)
    wd = W1a[2 * d:].astype(_BF)
    he_new, weighted = _tc_edge(
        he, g3, norm.reshape(e_total, 1), we, ws, wd,
        b1a.reshape(1, d), W1b.astype(_BF), b1b.reshape(1, d))

    acc = jax.new_ref(jnp.zeros((n, d), _F32))
    _sc_scatter_add(weighted, dst.reshape(1, e_total), acc)
    node_ftr = acc[...]

    wh = W2a[:d].astype(_BF)
    wf = W2a[d:].astype(_BF)
    hn_new = _tc_node(hn, node_ftr, wh, wf, b2a.reshape(1, d),
                      W2b.astype(_BF), b2b.reshape(1, d))
    return he_new, hn_new
